# baseline (speedup 1.0000x reference)
import jax
import jax.numpy as jnp
from jax.experimental import pallas as pl
from jax.experimental.pallas import tpu as pltpu

NUM_LEVELS_ = 8
K_ = 8192
D_ = 256
B_ = 4096
BT_ = 256
NBT_ = B_ // BT_
NT_ = NUM_LEVELS_ * NBT_

_HI = jax.lax.Precision.HIGHEST
_BIG = 3e38


def _nt(a, b):
    return jax.lax.dot_general(a, b, (((1,), (1,)), ((), ())),
                               preferred_element_type=jnp.float32)


def _nn(a, b):
    return jax.lax.dot_general(a, b, (((1,), (0,)), ((), ())),
                               preferred_element_type=jnp.float32)


def _split3(x):
    x0 = x.astype(jnp.bfloat16)
    rem = x - x0.astype(jnp.float32)
    x1 = rem.astype(jnp.bfloat16)
    x2 = (rem - x1.astype(jnp.float32)).astype(jnp.bfloat16)
    return x0, x1, x2


def _rvq_body(r_in_ref, cb_ref, ids_ref, r_out_ref,
              r_scratch, cnorm_scratch, c0_s, c1_s, c2_s, idx_s):
    t = pl.program_id(0)
    b = t % NBT_
    bp = (t + NT_ - 1) % NBT_

    def pick_row(idxv, kiota):
        onehot = (kiota == idxv[:, None]).astype(jnp.bfloat16)
        return (_nn(onehot, c0_s[...]) + _nn(onehot, c1_s[...])
                + _nn(onehot, c2_s[...]))

    def ref_dist(rp, crow):
        diff = rp - crow
        d2 = jnp.sum(diff * diff, axis=1)
        return jnp.sqrt(jnp.maximum(d2, 0.0))

    def extract_prev():
        idx1 = idx_s[(t + 1) % 2, 0, :]
        idx2 = idx_s[(t + 1) % 2, 1, :]
        kiota = jax.lax.broadcasted_iota(jnp.int32, (BT_, K_), 1)
        g1 = pick_row(idx1, kiota)
        g2 = pick_row(idx2, kiota)
        rp = r_scratch[pl.ds(bp * BT_, BT_), :]
        d1 = ref_dist(rp, g1)
        d2 = ref_dist(rp, g2)
        take2 = jnp.logical_or(d2 < d1,
                               jnp.logical_and(d2 == d1, idx2 < idx1))
        winner = jnp.where(take2, idx2, idx1)
        chosen = jnp.where(take2[:, None], g2, g1)
        r_new = rp - chosen
        r_scratch[pl.ds(bp * BT_, BT_), :] = r_new
        r_out_ref[...] = r_new
        ids_ref[...] = winner.reshape(1, 1, BT_)

    @pl.when(jnp.logical_and(t > 0, b == 0))
    def _extract_at_boundary():
        extract_prev()

    @pl.when(jnp.logical_and(b == 0, t < NT_))
    def _per_level_prep():
        cb = cb_ref[0]
        ones = jnp.ones((1, D_), jnp.float32)
        cnorm_scratch[...] = jax.lax.dot_general(
            ones, cb * cb, (((1,), (1,)), ((), ())),
            precision=_HI, preferred_element_type=jnp.float32)
        p0, p1, p2 = _split3(cb)
        c0_s[...] = p0
        c1_s[...] = p1
        c2_s[...] = p2

    @pl.when(t < NT_)
    def _scores_and_argmin():
        @pl.when(t < NBT_)
        def _init_residual():
            r_scratch[pl.ds(b * BT_, BT_), :] = r_in_ref[...]

        r = r_scratch[pl.ds(b * BT_, BT_), :]
        r0, r1, r2 = _split3(r)
        c0 = c0_s[...]
        c1 = c1_s[...]
        c2 = c2_s[...]
        rc = (_nt(r0, c0) + _nt(r0, c1) + _nt(r1, c0)
              + _nt(r0, c2) + _nt(r1, c1) + _nt(r2, c0))
        s = cnorm_scratch[...] - 2.0 * rc

        idx1 = jnp.argmin(s, axis=1).astype(jnp.int32)
        kiota = jax.lax.broadcasted_iota(jnp.int32, (BT_, K_), 1)
        s2 = jnp.where(kiota == idx1[:, None], _BIG, s)
        idx2 = jnp.argmin(s2, axis=1).astype(jnp.int32)
        idx_s[t % 2, 0, :] = idx1
        idx_s[t % 2, 1, :] = idx2

    @pl.when(jnp.logical_and(t > 0, b != 0))
    def _extract_overlapped():
        extract_prev()


def kernel(preference_vector, codebooks):
    ids_lb, residual = pl.pallas_call(
        _rvq_body,
        grid=(NT_ + 1,),
        in_specs=[
            pl.BlockSpec((BT_, D_), lambda t: (t % NBT_, 0)),
            pl.BlockSpec((1, K_, D_), lambda t: ((t % NT_) // NBT_, 0, 0)),
        ],
        out_specs=[
            pl.BlockSpec(
                (1, 1, BT_),
                lambda t: (((t + NT_ - 1) // NBT_) % NUM_LEVELS_, 0,
                           (t + NT_ - 1) % NBT_)),
            pl.BlockSpec((BT_, D_), lambda t: ((t + NT_ - 1) % NBT_, 0)),
        ],
        out_shape=[
            jax.ShapeDtypeStruct((NUM_LEVELS_, 1, B_), jnp.int32),
            jax.ShapeDtypeStruct((B_, D_), jnp.float32),
        ],
        scratch_shapes=[
            pltpu.VMEM((B_, D_), jnp.float32),
            pltpu.VMEM((1, K_), jnp.float32),
            pltpu.VMEM((K_, D_), jnp.bfloat16),
            pltpu.VMEM((K_, D_), jnp.bfloat16),
            pltpu.VMEM((K_, D_), jnp.bfloat16),
            pltpu.VMEM((2, 2, BT_), jnp.int32),
        ],
    )(preference_vector, codebooks)
    ids = ids_lb.reshape(NUM_LEVELS_, B_).T
    return ids, residual


# speedup vs baseline: 1.0981x; 1.0981x over previous
import jax
import jax.numpy as jnp
from jax.experimental import pallas as pl
from jax.experimental.pallas import tpu as pltpu

NUM_LEVELS_ = 8
K_ = 8192
D_ = 256
B_ = 4096
BT_ = 256
NBT_ = B_ // BT_
NT_ = NUM_LEVELS_ * NBT_

_HI = jax.lax.Precision.HIGHEST
_BIG = 3e38
_TAU = 1e-3


def _nt(a, b):
    return jax.lax.dot_general(a, b, (((1,), (1,)), ((), ())),
                               preferred_element_type=jnp.float32)


def _nn(a, b):
    return jax.lax.dot_general(a, b, (((1,), (0,)), ((), ())),
                               preferred_element_type=jnp.float32)


def _split3(x):
    x0 = x.astype(jnp.bfloat16)
    rem = x - x0.astype(jnp.float32)
    x1 = rem.astype(jnp.bfloat16)
    x2 = (rem - x1.astype(jnp.float32)).astype(jnp.bfloat16)
    return x0, x1, x2


def _rvq_body(r_in_ref, cb_ref, ids_ref, r_out_ref,
              r_scratch, cnorm_scratch, c0_s, c1_s, c2_s, idx_s, flag_s):
    t = pl.program_id(0)
    b = t % NBT_
    bp = (t + NT_ - 1) % NBT_

    def pick_row(idxv, kiota):
        onehot = (kiota == idxv[:, None]).astype(jnp.bfloat16)
        return (_nn(onehot, c0_s[...]) + _nn(onehot, c1_s[...])
                + _nn(onehot, c2_s[...]))

    def ref_dist(rp, crow):
        diff = rp - crow
        d2 = jnp.sum(diff * diff, axis=1)
        return jnp.sqrt(jnp.maximum(d2, 0.0))

    def extract_prev():
        idx1 = idx_s[(t + 1) % 2, 0, :]
        kiota = jax.lax.broadcasted_iota(jnp.int32, (BT_, K_), 1)
        g1 = pick_row(idx1, kiota)
        rp = r_scratch[pl.ds(bp * BT_, BT_), :]
        r_scratch[pl.ds(bp * BT_, BT_), :] = rp - g1
        r_out_ref[...] = rp - g1
        ids_ref[...] = idx1.reshape(1, 1, BT_)

        @pl.when(flag_s[(t + 1) % 2] != 0)
        def _rescore_near_ties():
            idx2 = idx_s[(t + 1) % 2, 1, :]
            g2 = pick_row(idx2, kiota)
            d1 = ref_dist(rp, g1)
            d2 = ref_dist(rp, g2)
            take2 = jnp.logical_or(d2 < d1,
                                   jnp.logical_and(d2 == d1, idx2 < idx1))
            winner = jnp.where(take2, idx2, idx1)
            chosen = jnp.where(take2[:, None], g2, g1)
            r_new = rp - chosen
            r_scratch[pl.ds(bp * BT_, BT_), :] = r_new
            r_out_ref[...] = r_new
            ids_ref[...] = winner.reshape(1, 1, BT_)

    @pl.when(jnp.logical_and(t > 0, b == 0))
    def _extract_at_boundary():
        extract_prev()

    @pl.when(jnp.logical_and(b == 0, t < NT_))
    def _per_level_prep():
        cb = cb_ref[0]
        ones = jnp.ones((1, D_), jnp.float32)
        cnorm_scratch[...] = jax.lax.dot_general(
            ones, cb * cb, (((1,), (1,)), ((), ())),
            precision=_HI, preferred_element_type=jnp.float32)
        p0, p1, p2 = _split3(cb)
        c0_s[...] = p0
        c1_s[...] = p1
        c2_s[...] = p2

    @pl.when(t < NT_)
    def _scores_and_argmin():
        @pl.when(t < NBT_)
        def _init_residual():
            r_scratch[pl.ds(b * BT_, BT_), :] = r_in_ref[...]

        r = r_scratch[pl.ds(b * BT_, BT_), :]
        r0, r1, r2 = _split3(r)
        c0 = c0_s[...]
        c1 = c1_s[...]
        c2 = c2_s[...]
        rc = (_nt(r0, c0) + _nt(r0, c1) + _nt(r1, c0)
              + _nt(r0, c2) + _nt(r1, c1) + _nt(r2, c0))
        s = cnorm_scratch[...] - 2.0 * rc

        idx1 = jnp.argmin(s, axis=1).astype(jnp.int32)
        m1 = jnp.min(s, axis=1, keepdims=True)
        kiota = jax.lax.broadcasted_iota(jnp.int32, (BT_, K_), 1)
        s2 = jnp.where(kiota == idx1[:, None], _BIG, s)
        idx2 = jnp.argmin(s2, axis=1).astype(jnp.int32)
        m2 = jnp.min(s2, axis=1, keepdims=True)
        idx_s[t % 2, 0, :] = idx1
        idx_s[t % 2, 1, :] = idx2
        near = jnp.any(m2 - m1 < _TAU)
        flag_s[t % 2] = near.astype(jnp.int32)

    @pl.when(jnp.logical_and(t > 0, b != 0))
    def _extract_overlapped():
        extract_prev()


def kernel(preference_vector, codebooks):
    ids_lb, residual = pl.pallas_call(
        _rvq_body,
        grid=(NT_ + 1,),
        in_specs=[
            pl.BlockSpec((BT_, D_), lambda t: (t % NBT_, 0)),
            pl.BlockSpec((1, K_, D_), lambda t: ((t % NT_) // NBT_, 0, 0)),
        ],
        out_specs=[
            pl.BlockSpec(
                (1, 1, BT_),
                lambda t: (((t + NT_ - 1) // NBT_) % NUM_LEVELS_, 0,
                           (t + NT_ - 1) % NBT_)),
            pl.BlockSpec((BT_, D_), lambda t: ((t + NT_ - 1) % NBT_, 0)),
        ],
        out_shape=[
            jax.ShapeDtypeStruct((NUM_LEVELS_, 1, B_), jnp.int32),
            jax.ShapeDtypeStruct((B_, D_), jnp.float32),
        ],
        scratch_shapes=[
            pltpu.VMEM((B_, D_), jnp.float32),
            pltpu.VMEM((1, K_), jnp.float32),
            pltpu.VMEM((K_, D_), jnp.bfloat16),
            pltpu.VMEM((K_, D_), jnp.bfloat16),
            pltpu.VMEM((K_, D_), jnp.bfloat16),
            pltpu.VMEM((2, 2, BT_), jnp.int32),
            pltpu.SMEM((2,), jnp.int32),
        ],
    )(preference_vector, codebooks)
    ids = ids_lb.reshape(NUM_LEVELS_, B_).T
    return ids, residual


# speedup vs baseline: 1.1231x; 1.0228x over previous
import jax
import jax.numpy as jnp
from jax.experimental import pallas as pl
from jax.experimental.pallas import tpu as pltpu

NUM_LEVELS_ = 8
K_ = 8192
D_ = 256
B_ = 4096
BT_ = 256
NBT_ = B_ // BT_
NT_ = NUM_LEVELS_ * NBT_

_HI = jax.lax.Precision.HIGHEST
_BIG = 3e38
_TAU = 1e-3


def _nt(a, b):
    return jax.lax.dot_general(a, b, (((1,), (1,)), ((), ())),
                               preferred_element_type=jnp.float32)


def _nn(a, b):
    return jax.lax.dot_general(a, b, (((1,), (0,)), ((), ())),
                               preferred_element_type=jnp.float32)


def _split3(x):
    x0 = x.astype(jnp.bfloat16)
    rem = x - x0.astype(jnp.float32)
    x1 = rem.astype(jnp.bfloat16)
    x2 = (rem - x1.astype(jnp.float32)).astype(jnp.bfloat16)
    return x0, x1, x2


def _rvq_body(r_in_ref, cb_ref, ids_ref, r_out_ref,
              r_scratch, cnorm_scratch, c0_s, c1_s, c2_s, idx_s, flag_s):
    t = pl.program_id(0)
    b = t % NBT_
    bp = (t + NT_ - 1) % NBT_

    def pick_row(idxv, kiota):
        onehot = (kiota == idxv[:, None]).astype(jnp.bfloat16)
        return (_nn(onehot, c0_s[...]) + _nn(onehot, c1_s[...])
                + _nn(onehot, c2_s[...]))

    def ref_dist(rp, crow):
        diff = rp - crow
        d2 = jnp.sum(diff * diff, axis=1)
        return jnp.sqrt(jnp.maximum(d2, 0.0))

    def extract_prev():
        idx1 = idx_s[(t + 1) % 2, 0, :]
        kiota = jax.lax.broadcasted_iota(jnp.int32, (BT_, K_), 1)
        g1 = pick_row(idx1, kiota)
        rp = r_scratch[pl.ds(bp * BT_, BT_), :]
        r_scratch[pl.ds(bp * BT_, BT_), :] = rp - g1
        r_out_ref[...] = rp - g1
        ids_ref[...] = idx1.reshape(1, 1, BT_)

        @pl.when(flag_s[(t + 1) % 2] != 0)
        def _rescore_near_ties():
            idx2 = idx_s[(t + 1) % 2, 1, :]
            g2 = pick_row(idx2, kiota)
            d1 = ref_dist(rp, g1)
            d2 = ref_dist(rp, g2)
            take2 = jnp.logical_or(d2 < d1,
                                   jnp.logical_and(d2 == d1, idx2 < idx1))
            winner = jnp.where(take2, idx2, idx1)
            chosen = jnp.where(take2[:, None], g2, g1)
            r_new = rp - chosen
            r_scratch[pl.ds(bp * BT_, BT_), :] = r_new
            r_out_ref[...] = r_new
            ids_ref[...] = winner.reshape(1, 1, BT_)

    @pl.when(jnp.logical_and(t > 0, b == 0))
    def _extract_at_boundary():
        extract_prev()

    @pl.when(jnp.logical_and(b == 0, t < NT_))
    def _per_level_prep():
        cb = cb_ref[0]
        ones = jnp.ones((1, D_), jnp.float32)
        cnorm_scratch[...] = jax.lax.dot_general(
            ones, cb * cb, (((1,), (1,)), ((), ())),
            precision=_HI, preferred_element_type=jnp.float32)
        p0, p1, p2 = _split3(cb)
        c0_s[...] = p0
        c1_s[...] = p1
        c2_s[...] = p2

    @pl.when(t < NT_)
    def _scores_and_argmin():
        @pl.when(t < NBT_)
        def _init_residual():
            r_scratch[pl.ds(b * BT_, BT_), :] = r_in_ref[...]

        r = r_scratch[pl.ds(b * BT_, BT_), :]
        r0, r1, r2 = _split3(r)
        c0 = c0_s[...]
        c1 = c1_s[...]
        c2 = c2_s[...]
        rc = (_nt(r0, c0) + _nt(r0, c1) + _nt(r1, c0)
              + _nt(r0, c2) + _nt(r1, c1) + _nt(r2, c0))
        s = cnorm_scratch[...] - 2.0 * rc

        m1 = jnp.min(s, axis=1, keepdims=True)
        kiota = jax.lax.broadcasted_iota(jnp.int32, (BT_, K_), 1)
        idx1 = jnp.min(jnp.where(s == m1, kiota, K_), axis=1)
        idx_s[t % 2, 0, :] = idx1
        cnt = jnp.sum((s < m1 + _TAU).astype(jnp.int32))
        near = cnt > BT_
        flag_s[t % 2] = near.astype(jnp.int32)

        @pl.when(near)
        def _runner_up():
            s2 = jnp.where(kiota == idx1[:, None], _BIG, s)
            idx2 = jnp.argmin(s2, axis=1).astype(jnp.int32)
            idx_s[t % 2, 1, :] = idx2

    @pl.when(jnp.logical_and(t > 0, b != 0))
    def _extract_overlapped():
        extract_prev()


def kernel(preference_vector, codebooks):
    ids_lb, residual = pl.pallas_call(
        _rvq_body,
        grid=(NT_ + 1,),
        in_specs=[
            pl.BlockSpec((BT_, D_), lambda t: (t % NBT_, 0)),
            pl.BlockSpec((1, K_, D_), lambda t: ((t % NT_) // NBT_, 0, 0)),
        ],
        out_specs=[
            pl.BlockSpec(
                (1, 1, BT_),
                lambda t: (((t + NT_ - 1) // NBT_) % NUM_LEVELS_, 0,
                           (t + NT_ - 1) % NBT_)),
            pl.BlockSpec((BT_, D_), lambda t: ((t + NT_ - 1) % NBT_, 0)),
        ],
        out_shape=[
            jax.ShapeDtypeStruct((NUM_LEVELS_, 1, B_), jnp.int32),
            jax.ShapeDtypeStruct((B_, D_), jnp.float32),
        ],
        scratch_shapes=[
            pltpu.VMEM((B_, D_), jnp.float32),
            pltpu.VMEM((1, K_), jnp.float32),
            pltpu.VMEM((K_, D_), jnp.bfloat16),
            pltpu.VMEM((K_, D_), jnp.bfloat16),
            pltpu.VMEM((K_, D_), jnp.bfloat16),
            pltpu.VMEM((2, 2, BT_), jnp.int32),
            pltpu.SMEM((2,), jnp.int32),
        ],
    )(preference_vector, codebooks)
    ids = ids_lb.reshape(NUM_LEVELS_, B_).T
    return ids, residual


# speedup vs baseline: 1.7958x; 1.5989x over previous
import functools
import jax
import jax.numpy as jnp
from jax import lax
from jax.experimental import pallas as pl
from jax.experimental.pallas import tpu as pltpu

try:
    from jax.experimental.pallas import tpu_sc as plsc
except ImportError:
    plsc = None

NUM_LEVELS_ = 8
K_ = 8192
D_ = 256
B_ = 4096
BT_ = 256
NBT_ = B_ // BT_

_HI = jax.lax.Precision.HIGHEST


def _nt(a, b):
    return jax.lax.dot_general(a, b, (((1,), (1,)), ((), ())),
                               preferred_element_type=jnp.float32)


def _split3(x):
    x0 = x.astype(jnp.bfloat16)
    rem = x - x0.astype(jnp.float32)
    x1 = rem.astype(jnp.bfloat16)
    x2 = (rem - x1.astype(jnp.float32)).astype(jnp.bfloat16)
    return x0, x1, x2


def _scores_body(r_ref, cb_ref, ids_ref, cnorm_scratch, c0_s, c1_s, c2_s):
    b = pl.program_id(0)

    @pl.when(b == 0)
    def _prep():
        cb = cb_ref[...]
        ones = jnp.ones((1, D_), jnp.float32)
        cnorm_scratch[...] = jax.lax.dot_general(
            ones, cb * cb, (((1,), (1,)), ((), ())),
            precision=_HI, preferred_element_type=jnp.float32)
        p0, p1, p2 = _split3(cb)
        c0_s[...] = p0
        c1_s[...] = p1
        c2_s[...] = p2

    r = r_ref[...]
    r0, r1, r2 = _split3(r)
    c0 = c0_s[...]
    c1 = c1_s[...]
    c2 = c2_s[...]
    rc = (_nt(r0, c0) + _nt(r0, c1) + _nt(r1, c0)
          + _nt(r0, c2) + _nt(r1, c1) + _nt(r2, c0))
    s = cnorm_scratch[...] - 2.0 * rc
    idx = jnp.argmin(s, axis=1).astype(jnp.int32)
    ids_ref[...] = idx.reshape(1, 1, BT_)


def _tc_scores_argmin(r, cb_l):
    ids = pl.pallas_call(
        _scores_body,
        grid=(NBT_,),
        in_specs=[
            pl.BlockSpec((BT_, D_), lambda b: (b, 0)),
            pl.BlockSpec((K_, D_), lambda b: (0, 0)),
        ],
        out_specs=pl.BlockSpec((1, 1, BT_), lambda b: (0, 0, b)),
        out_shape=jax.ShapeDtypeStruct((1, 1, B_), jnp.int32),
        scratch_shapes=[
            pltpu.VMEM((1, K_), jnp.float32),
            pltpu.VMEM((K_, D_), jnp.bfloat16),
            pltpu.VMEM((K_, D_), jnp.bfloat16),
            pltpu.VMEM((K_, D_), jnp.bfloat16),
        ],
    )(r, cb_l)
    return ids.reshape(B_)


def _make_sc_gather_sub():
    info = plsc.get_sparse_core_info()
    nw = info.num_cores * info.num_subcores
    bw = B_ // nw
    mesh = plsc.VectorSubcoreMesh(core_axis_name="c", subcore_axis_name="s")

    @functools.partial(
        pl.kernel, mesh=mesh,
        out_type=jax.ShapeDtypeStruct((B_, D_), jnp.float32),
        scratch_types=[
            pltpu.VMEM((bw,), jnp.int32),
            pltpu.VMEM((bw, D_), jnp.float32),
            pltpu.VMEM((bw, D_), jnp.float32),
            pltpu.SemaphoreType.DMA,
        ],
    )
    def sc_gather_sub(cb_hbm, idx_hbm, r_hbm, out_hbm, idx_v, rows_v, r_v, sem):
        wid = lax.axis_index("s") * info.num_cores + lax.axis_index("c")
        base = wid * bw
        pltpu.sync_copy(idx_hbm.at[pl.ds(base, bw)], idx_v)
        cp = pltpu.async_copy(cb_hbm.at[idx_v], rows_v, sem)
        pltpu.sync_copy(r_hbm.at[pl.ds(base, bw)], r_v)
        cp.wait()

        def sub_row(i, carry):
            for j in range(D_ // 16):
                sl = pl.ds(j * 16, 16)
                r_v[i, sl] = r_v[i, sl] - rows_v[i, sl]
            return carry

        lax.fori_loop(0, bw, sub_row, 0)
        pltpu.sync_copy(r_v, out_hbm.at[pl.ds(base, bw)])

    return sc_gather_sub


def kernel(preference_vector, codebooks):
    sc_gather_sub = _make_sc_gather_sub()
    r = preference_vector
    ids = []
    for l in range(NUM_LEVELS_):
        cb_l = codebooks[l]
        idx_l = _tc_scores_argmin(r, cb_l)
        r = sc_gather_sub(cb_l, idx_l, r)
        ids.append(idx_l)
    return jnp.stack(ids, axis=1), r
